# Initial kernel scaffold; baseline (speedup 1.0000x reference)
import jax
import jax.numpy as jnp
from jax.experimental import pallas as pl


def kernel(x, edge_index, Wl1, Wr1, b1, Wl2, Wr2, b2):
    raise NotImplementedError("write your pallas kernel here")


# speedup vs baseline: 3.4236x; 3.4236x over previous
import functools

import jax
import jax.numpy as jnp
from jax import lax
from jax.experimental import pallas as pl
from jax.experimental.pallas import tpu as pltpu
from jax.experimental.pallas import tpu_sc as plsc

N = 10000
C = 128
CH = C // 2
E = 320000
NC, NS = 2, 16
E_PER_TILE = E // NS
CHUNK = 80
NCHUNK = E_PER_TILE // CHUNK
NPAD = 10240
ROWS_PER_TILE = NPAD // NS
CW = 16


def _segsum_body(do_counts, src_hbm, dst_hbm, xab_hbm, z2d_hbm, zc_hbm,
                 ones_hbm, *rest):
    if do_counts:
        out_hbm, cnt_hbm, acc, cacc, src_v, dst_v, rows_v, ones_v, cb_v, sem = rest
    else:
        out_hbm, acc, src_v, dst_v, rows_v, sem = rest
        cnt_hbm = cacc = ones_v = cb_v = None

    c = lax.axis_index("c")
    s = lax.axis_index("s")
    r_base = pl.multiple_of(s * ROWS_PER_TILE, ROWS_PER_TILE)

    pltpu.sync_copy(z2d_hbm, rows_v)
    for k in range(ROWS_PER_TILE // CHUNK):
        pltpu.sync_copy(rows_v, acc.at[pl.ds(r_base + k * CHUNK, CHUNK)])
    if do_counts:
        pltpu.sync_copy(zc_hbm, cb_v)
        pltpu.sync_copy(cb_v, cacc.at[pl.ds(r_base, ROWS_PER_TILE)])
        pltpu.sync_copy(ones_hbm, ones_v)
    plsc.subcore_barrier()

    row_off = jnp.full((16,), c * N, jnp.int32)

    def chunk_body(j, carry):
        base = pl.multiple_of(s * E_PER_TILE + j * CHUNK, CHUNK)
        pltpu.sync_copy(src_hbm.at[pl.ds(base, CHUNK)], src_v)
        pltpu.sync_copy(dst_hbm.at[pl.ds(base, CHUNK)], dst_v)
        for i in range(CHUNK // 16):
            sl = pl.ds(i * 16, 16)
            src_v[sl] = src_v[sl] + row_off
        pltpu.async_copy(xab_hbm.at[src_v], rows_v, sem).wait()
        pltpu.sync_copy(rows_v, acc.at[dst_v], add=True)
        if do_counts:
            @pl.when(c == 0)
            def _():
                pltpu.sync_copy(ones_v, cacc.at[dst_v], add=True)
        return carry

    lax.fori_loop(0, NCHUNK, chunk_body, 0)
    plsc.subcore_barrier()

    for k in range(ROWS_PER_TILE // CHUNK):
        r0 = r_base + k * CHUNK
        pltpu.sync_copy(acc.at[pl.ds(r0, CHUNK)], rows_v)
        pltpu.sync_copy(rows_v, out_hbm.at[c, pl.ds(r0, CHUNK)])
    if do_counts:
        pltpu.sync_copy(cacc.at[pl.ds(r_base, ROWS_PER_TILE)], cb_v)
        pltpu.sync_copy(cb_v, cnt_hbm.at[c, pl.ds(r_base, ROWS_PER_TILE)])


def _make_segsum(do_counts):
    out_type = [jax.ShapeDtypeStruct((NC, NPAD, CH), jnp.float32)]
    scratch = [pltpu.VMEM_SHARED((NPAD, CH), jnp.float32)]
    if do_counts:
        out_type.append(jax.ShapeDtypeStruct((NC, NPAD, CW), jnp.float32))
        scratch.append(pltpu.VMEM_SHARED((NPAD, CW), jnp.float32))
    scratch += [
        pltpu.VMEM((CHUNK,), jnp.int32),
        pltpu.VMEM((CHUNK,), jnp.int32),
        pltpu.VMEM((CHUNK, CH), jnp.float32),
    ]
    if do_counts:
        scratch.append(pltpu.VMEM((CHUNK, CW), jnp.float32))
        scratch.append(pltpu.VMEM((ROWS_PER_TILE, CW), jnp.float32))
    scratch.append(pltpu.SemaphoreType.DMA)
    mesh = plsc.VectorSubcoreMesh(core_axis_name="c", subcore_axis_name="s")
    return pl.kernel(
        functools.partial(_segsum_body, do_counts),
        out_type=tuple(out_type),
        mesh=mesh,
        scratch_types=scratch,
        compiler_params=pltpu.CompilerParams(use_tc_tiling_on_sc=False),
    )


_BM = 1024


def _tc1_body(p_ref, cntp_ref, x_ref, wl_ref, wr_ref, b_ref, h_ref, hab_ref):
    cnt = cntp_ref[0, :, 0]
    inv = 1.0 / jnp.maximum(cnt, 1.0)
    mean = jnp.concatenate([p_ref[0], p_ref[1]], axis=1) * inv[:, None]
    h = (jnp.dot(mean, wl_ref[...], preferred_element_type=jnp.float32)
         + jnp.dot(x_ref[...], wr_ref[...], preferred_element_type=jnp.float32)
         + b_ref[...])
    h = jnp.maximum(h, 0.0)
    h_ref[...] = h
    hab_ref[0] = h[:, :CH]
    hab_ref[1] = h[:, CH:]


def _tc2_body(p_ref, cntp_ref, h_ref, wl_ref, wr_ref, b_ref, o_ref):
    cnt = cntp_ref[0, :, 0]
    inv = 1.0 / jnp.maximum(cnt, 1.0)
    mean = jnp.concatenate([p_ref[0], p_ref[1]], axis=1) * inv[:, None]
    o_ref[...] = (jnp.dot(mean, wl_ref[...], preferred_element_type=jnp.float32)
                  + jnp.dot(h_ref[...], wr_ref[...], preferred_element_type=jnp.float32)
                  + b_ref[...])


def _tc_layer1(p, cntp, x, wl, wr, b):
    grid = (pl.cdiv(N, _BM),)
    return pl.pallas_call(
        _tc1_body,
        grid=grid,
        in_specs=[
            pl.BlockSpec((NC, _BM, CH), lambda i: (0, i, 0)),
            pl.BlockSpec((NC, _BM, CW), lambda i: (0, i, 0)),
            pl.BlockSpec((_BM, C), lambda i: (i, 0)),
            pl.BlockSpec((C, C), lambda i: (0, 0)),
            pl.BlockSpec((C, C), lambda i: (0, 0)),
            pl.BlockSpec((1, C), lambda i: (0, 0)),
        ],
        out_specs=[
            pl.BlockSpec((_BM, C), lambda i: (i, 0)),
            pl.BlockSpec((NC, _BM, CH), lambda i: (0, i, 0)),
        ],
        out_shape=[
            jax.ShapeDtypeStruct((N, C), jnp.float32),
            jax.ShapeDtypeStruct((NC, N, CH), jnp.float32),
        ],
    )(p, cntp, x, wl, wr, b)


def _tc_layer2(p, cntp, h, wl, wr, b):
    grid = (pl.cdiv(N, _BM),)
    return pl.pallas_call(
        _tc2_body,
        grid=grid,
        in_specs=[
            pl.BlockSpec((NC, _BM, CH), lambda i: (0, i, 0)),
            pl.BlockSpec((NC, _BM, CW), lambda i: (0, i, 0)),
            pl.BlockSpec((_BM, C), lambda i: (i, 0)),
            pl.BlockSpec((C, C), lambda i: (0, 0)),
            pl.BlockSpec((C, C), lambda i: (0, 0)),
            pl.BlockSpec((1, C), lambda i: (0, 0)),
        ],
        out_specs=pl.BlockSpec((_BM, C), lambda i: (i, 0)),
        out_shape=jax.ShapeDtypeStruct((N, C), jnp.float32),
    )(p, cntp, h, wl, wr, b)


@jax.jit
def kernel(x, edge_index, Wl1, Wr1, b1, Wl2, Wr2, b2):
    src = edge_index[0].astype(jnp.int32)
    dst = edge_index[1].astype(jnp.int32)
    z2d = jnp.zeros((CHUNK, CH), jnp.float32)
    zc = jnp.zeros((ROWS_PER_TILE, CW), jnp.float32)
    ones = jnp.ones((CHUNK, CW), jnp.float32)

    xab = jnp.concatenate([x[:, :CH], x[:, CH:]], axis=0)
    p1, cntp = _make_segsum(True)(src, dst, xab, z2d, zc, ones)
    h, hab = _tc_layer1(p1, cntp, x, Wl1, Wr1, b1.reshape(1, C))
    (p2,) = _make_segsum(False)(src, dst, hab.reshape(NC * N, CH), z2d, zc, ones)
    out = _tc_layer2(p2, cntp, h, Wl2, Wr2, b2.reshape(1, C))
    return out


# speedup vs baseline: 10.2254x; 2.9867x over previous
import functools

import jax
import jax.numpy as jnp
from jax import lax
from jax.experimental import pallas as pl
from jax.experimental.pallas import tpu as pltpu
from jax.experimental.pallas import tpu_sc as plsc

N = 10000
C = 128
CH = C // 2
E = 320000
NC, NS = 2, 16
E_PER_TILE = E // NS
CHUNK = 40
NCHUNK = E_PER_TILE // CHUNK
NPAD = 10240
ROWS_PER_TILE = NPAD // NS
CW = 16


NBUF = 10
LAG = NBUF // 2
NSUPER = NCHUNK // NBUF


def _segsum_body(do_counts, srcab_hbm, dst_hbm, xab_hbm, z2d_hbm, zc_hbm,
                 ones_hbm, *rest):
    if do_counts:
        out_hbm, cnt_hbm, acc, cacc, src_a, dst_a, ones_v, *rows = rest
    else:
        out_hbm, acc, src_a, dst_a, *rows = rest
        cnt_hbm = cacc = ones_v = None
    rows, sems = rows[:NBUF], rows[NBUF:]
    gsem, ssem = sems[:NBUF], sems[NBUF:2 * NBUF]
    csem = sems[2 * NBUF] if do_counts else None

    c = lax.axis_index("c")
    s = lax.axis_index("s")
    r_base = pl.multiple_of(s * ROWS_PER_TILE, ROWS_PER_TILE)

    pltpu.sync_copy(srcab_hbm.at[c, s], src_a)
    pltpu.sync_copy(dst_hbm.at[s], dst_a)

    pltpu.sync_copy(z2d_hbm, rows[0])
    for k in range(ROWS_PER_TILE // CHUNK):
        pltpu.sync_copy(rows[0], acc.at[pl.ds(r_base + k * CHUNK, CHUNK)])
    if do_counts:
        pltpu.sync_copy(zc_hbm, ones_v)
        for k in range(ROWS_PER_TILE // CHUNK):
            pltpu.sync_copy(ones_v, cacc.at[pl.ds(r_base + k * CHUNK, CHUNK)])
        pltpu.sync_copy(ones_hbm, ones_v)

    for b in range(LAG):
        pltpu.async_copy(xab_hbm.at[src_a.at[b]], rows[b], gsem[b])
    plsc.subcore_barrier()

    def super_body(J, carry):
        for b in range(NBUF):
            j = J * NBUF + b
            pltpu.make_async_copy(xab_hbm.at[src_a.at[j]], rows[b],
                                  gsem[b]).wait()
            pltpu.async_copy(rows[b], acc.at[dst_a.at[j]], ssem[b], add=True)
            if do_counts:
                @pl.when(c == 0)
                def _():
                    if b == 0:
                        @pl.when(J > 0)
                        def _():
                            pltpu.make_async_copy(
                                ones_v, cacc.at[dst_a.at[j]], csem).wait()
                    else:
                        pltpu.make_async_copy(
                            ones_v, cacc.at[dst_a.at[j]], csem).wait()
                    pltpu.async_copy(ones_v, cacc.at[dst_a.at[j]], csem,
                                     add=True)
            bp = (b + LAG) % NBUF
            jp = j - LAG
            jn = j + LAG

            def drain_and_refill():
                pltpu.make_async_copy(rows[bp], acc.at[dst_a.at[jp]],
                                      ssem[bp]).wait()
                pltpu.async_copy(xab_hbm.at[src_a.at[jn]], rows[bp], gsem[bp])

            if b < LAG:
                @pl.when(J > 0)
                def _():
                    drain_and_refill()

                @pl.when(J == 0)
                def _():
                    pltpu.async_copy(xab_hbm.at[src_a.at[jn]], rows[bp],
                                     gsem[bp])
            else:
                @pl.when(J < NSUPER - 1)
                def _():
                    drain_and_refill()

                @pl.when(J == NSUPER - 1)
                def _():
                    pltpu.make_async_copy(rows[bp], acc.at[dst_a.at[jp]],
                                          ssem[bp]).wait()
        return carry

    lax.fori_loop(0, NSUPER, super_body, 0)

    for b in range(LAG, NBUF):
        pltpu.make_async_copy(rows[b], acc.at[dst_a.at[NCHUNK - LAG + b - LAG]],
                              ssem[b]).wait()
    if do_counts:
        @pl.when(c == 0)
        def _():
            pltpu.make_async_copy(ones_v, cacc.at[dst_a.at[0]], csem).wait()
    plsc.subcore_barrier()

    for k in range(ROWS_PER_TILE // CHUNK):
        r0 = r_base + k * CHUNK
        pltpu.sync_copy(acc.at[pl.ds(r0, CHUNK)], rows[0])
        pltpu.sync_copy(rows[0], out_hbm.at[c, pl.ds(r0, CHUNK)])
    if do_counts:
        for k in range(ROWS_PER_TILE // CHUNK):
            r0 = r_base + k * CHUNK
            pltpu.sync_copy(cacc.at[pl.ds(r0, CHUNK)], ones_v)
            pltpu.sync_copy(ones_v, cnt_hbm.at[c, pl.ds(r0, CHUNK)])


def _make_segsum(do_counts):
    out_type = [jax.ShapeDtypeStruct((NC, NPAD, CH), jnp.float32)]
    scratch = [pltpu.VMEM_SHARED((NPAD, CH), jnp.float32)]
    if do_counts:
        out_type.append(jax.ShapeDtypeStruct((NC, NPAD, CW), jnp.float32))
        scratch.append(pltpu.VMEM_SHARED((NPAD, CW), jnp.float32))
    scratch += [
        pltpu.VMEM((NCHUNK, CHUNK), jnp.int32),
        pltpu.VMEM((NCHUNK, CHUNK), jnp.int32),
    ]
    if do_counts:
        scratch.append(pltpu.VMEM((CHUNK, CW), jnp.float32))
    scratch += [pltpu.VMEM((CHUNK, CH), jnp.float32) for _ in range(NBUF)]
    scratch += [pltpu.SemaphoreType.DMA for _ in range(2 * NBUF)]
    if do_counts:
        scratch.append(pltpu.SemaphoreType.DMA)
    mesh = plsc.VectorSubcoreMesh(core_axis_name="c", subcore_axis_name="s")
    return pl.kernel(
        functools.partial(_segsum_body, do_counts),
        out_type=tuple(out_type),
        mesh=mesh,
        scratch_types=scratch,
        compiler_params=pltpu.CompilerParams(use_tc_tiling_on_sc=False),
    )


_BM = 1024


def _tc1_body(p_ref, cntp_ref, x_ref, wl_ref, wr_ref, b_ref, h_ref, hab_ref):
    cnt = cntp_ref[0, :, 0]
    inv = 1.0 / jnp.maximum(cnt, 1.0)
    mean = jnp.concatenate([p_ref[0], p_ref[1]], axis=1) * inv[:, None]
    h = (jnp.dot(mean, wl_ref[...], preferred_element_type=jnp.float32)
         + jnp.dot(x_ref[...], wr_ref[...], preferred_element_type=jnp.float32)
         + b_ref[...])
    h = jnp.maximum(h, 0.0)
    h_ref[...] = h
    hab_ref[0] = h[:, :CH]
    hab_ref[1] = h[:, CH:]


def _tc2_body(p_ref, cntp_ref, h_ref, wl_ref, wr_ref, b_ref, o_ref):
    cnt = cntp_ref[0, :, 0]
    inv = 1.0 / jnp.maximum(cnt, 1.0)
    mean = jnp.concatenate([p_ref[0], p_ref[1]], axis=1) * inv[:, None]
    o_ref[...] = (jnp.dot(mean, wl_ref[...], preferred_element_type=jnp.float32)
                  + jnp.dot(h_ref[...], wr_ref[...], preferred_element_type=jnp.float32)
                  + b_ref[...])


def _tc_layer1(p, cntp, x, wl, wr, b):
    grid = (pl.cdiv(N, _BM),)
    return pl.pallas_call(
        _tc1_body,
        grid=grid,
        in_specs=[
            pl.BlockSpec((NC, _BM, CH), lambda i: (0, i, 0)),
            pl.BlockSpec((NC, _BM, CW), lambda i: (0, i, 0)),
            pl.BlockSpec((_BM, C), lambda i: (i, 0)),
            pl.BlockSpec((C, C), lambda i: (0, 0)),
            pl.BlockSpec((C, C), lambda i: (0, 0)),
            pl.BlockSpec((1, C), lambda i: (0, 0)),
        ],
        out_specs=[
            pl.BlockSpec((_BM, C), lambda i: (i, 0)),
            pl.BlockSpec((NC, _BM, CH), lambda i: (0, i, 0)),
        ],
        out_shape=[
            jax.ShapeDtypeStruct((N, C), jnp.float32),
            jax.ShapeDtypeStruct((NC, N, CH), jnp.float32),
        ],
    )(p, cntp, x, wl, wr, b)


def _tc_layer2(p, cntp, h, wl, wr, b):
    grid = (pl.cdiv(N, _BM),)
    return pl.pallas_call(
        _tc2_body,
        grid=grid,
        in_specs=[
            pl.BlockSpec((NC, _BM, CH), lambda i: (0, i, 0)),
            pl.BlockSpec((NC, _BM, CW), lambda i: (0, i, 0)),
            pl.BlockSpec((_BM, C), lambda i: (i, 0)),
            pl.BlockSpec((C, C), lambda i: (0, 0)),
            pl.BlockSpec((C, C), lambda i: (0, 0)),
            pl.BlockSpec((1, C), lambda i: (0, 0)),
        ],
        out_specs=pl.BlockSpec((_BM, C), lambda i: (i, 0)),
        out_shape=jax.ShapeDtypeStruct((N, C), jnp.float32),
    )(p, cntp, h, wl, wr, b)


@jax.jit
def kernel(x, edge_index, Wl1, Wr1, b1, Wl2, Wr2, b2):
    src = edge_index[0].astype(jnp.int32)
    dst = edge_index[1].astype(jnp.int32)
    z2d = jnp.zeros((CHUNK, CH), jnp.float32)
    zc = jnp.zeros((CHUNK, CW), jnp.float32)
    ones = jnp.ones((CHUNK, CW), jnp.float32)

    srcab = jnp.stack([src, src + N]).reshape(NC, NS, NCHUNK, CHUNK)
    dst4 = dst.reshape(NS, NCHUNK, CHUNK)
    xab = jnp.concatenate([x[:, :CH], x[:, CH:]], axis=0)
    p1, cntp = _make_segsum(True)(srcab, dst4, xab, z2d, zc, ones)
    h, hab = _tc_layer1(p1, cntp, x, Wl1, Wr1, b1.reshape(1, C))
    (p2,) = _make_segsum(False)(srcab, dst4, hab.reshape(NC * N, CH),
                                z2d, zc, ones)
    out = _tc_layer2(p2, cntp, h, Wl2, Wr2, b2.reshape(1, C))
    return out


# speedup vs baseline: 10.4340x; 1.0204x over previous
import functools

import jax
import jax.numpy as jnp
from jax import lax
from jax.experimental import pallas as pl
from jax.experimental.pallas import tpu as pltpu
from jax.experimental.pallas import tpu_sc as plsc

N = 10000
C = 128
CH = C // 2
E = 320000
NC, NS = 2, 16
E_PER_TILE = E // NS
CHUNK = 40
NCHUNK = E_PER_TILE // CHUNK
NPAD = 10240
ROWS_PER_TILE = NPAD // NS
CW = 16


NBUF = 10
LAG = NBUF // 2
NSUPER = NCHUNK // NBUF


def _segsum_body(do_counts, srcab_hbm, dst_hbm, xab_hbm, z2d_hbm, zc_hbm,
                 ones_hbm, *rest):
    if do_counts:
        out_hbm, cnt_hbm, acc, cacc, src_a, dst_a, ones_v, *rows = rest
    else:
        out_hbm, acc, src_a, dst_a, *rows = rest
        cnt_hbm = cacc = ones_v = None
    rows, sems = rows[:NBUF], rows[NBUF:]
    gsem, ssem = sems[:NBUF], sems[NBUF:2 * NBUF]
    csem = sems[2 * NBUF] if do_counts else None

    c = lax.axis_index("c")
    s = lax.axis_index("s")
    r_base = pl.multiple_of(s * ROWS_PER_TILE, ROWS_PER_TILE)

    pltpu.sync_copy(srcab_hbm.at[c, s], src_a)
    pltpu.sync_copy(dst_hbm.at[s], dst_a)

    pltpu.sync_copy(z2d_hbm, rows[0])
    if do_counts:
        pltpu.sync_copy(zc_hbm, ones_v)
    for k in range(ROWS_PER_TILE // CHUNK):
        pltpu.async_copy(rows[0], acc.at[pl.ds(r_base + k * CHUNK, CHUNK)],
                         ssem[0])
        if do_counts:
            pltpu.async_copy(ones_v,
                             cacc.at[pl.ds(r_base + k * CHUNK, CHUNK)],
                             ssem[1])
    for k in range(ROWS_PER_TILE // CHUNK):
        pltpu.make_async_copy(rows[0], acc.at[pl.ds(r_base, CHUNK)],
                              ssem[0]).wait()
        if do_counts:
            pltpu.make_async_copy(ones_v, cacc.at[pl.ds(r_base, CHUNK)],
                                  ssem[1]).wait()
    if do_counts:
        pltpu.sync_copy(ones_hbm, ones_v)

    for b in range(LAG):
        pltpu.async_copy(xab_hbm.at[src_a.at[b]], rows[b], gsem[b])
    plsc.subcore_barrier()

    def super_body(J, carry):
        for b in range(NBUF):
            j = J * NBUF + b
            pltpu.make_async_copy(xab_hbm.at[src_a.at[j]], rows[b],
                                  gsem[b]).wait()
            pltpu.async_copy(rows[b], acc.at[dst_a.at[j]], ssem[b], add=True)
            if do_counts:
                w0 = c * (NSUPER // 2)

                @pl.when((J >= w0) & (J < w0 + NSUPER // 2))
                def _():
                    if b == 0:
                        @pl.when(J > w0)
                        def _():
                            pltpu.make_async_copy(
                                ones_v, cacc.at[dst_a.at[j]], csem).wait()
                    else:
                        pltpu.make_async_copy(
                            ones_v, cacc.at[dst_a.at[j]], csem).wait()
                    pltpu.async_copy(ones_v, cacc.at[dst_a.at[j]], csem,
                                     add=True)
            bp = (b + LAG) % NBUF
            jp = j - LAG
            jn = j + LAG

            def drain_and_refill():
                pltpu.make_async_copy(rows[bp], acc.at[dst_a.at[jp]],
                                      ssem[bp]).wait()
                pltpu.async_copy(xab_hbm.at[src_a.at[jn]], rows[bp], gsem[bp])

            if b < LAG:
                @pl.when(J > 0)
                def _():
                    drain_and_refill()

                @pl.when(J == 0)
                def _():
                    pltpu.async_copy(xab_hbm.at[src_a.at[jn]], rows[bp],
                                     gsem[bp])
            else:
                @pl.when(J < NSUPER - 1)
                def _():
                    drain_and_refill()

                @pl.when(J == NSUPER - 1)
                def _():
                    pltpu.make_async_copy(rows[bp], acc.at[dst_a.at[jp]],
                                          ssem[bp]).wait()
        return carry

    lax.fori_loop(0, NSUPER, super_body, 0)

    for b in range(LAG, NBUF):
        pltpu.make_async_copy(rows[b], acc.at[dst_a.at[NCHUNK - LAG + b - LAG]],
                              ssem[b]).wait()
    if do_counts:
        pltpu.make_async_copy(ones_v, cacc.at[dst_a.at[0]], csem).wait()
    plsc.subcore_barrier()

    nk = ROWS_PER_TILE // CHUNK
    for k in range(nk):
        r0 = r_base + k * CHUNK
        if k >= 2:
            rp = r_base + (k - 2) * CHUNK
            pltpu.make_async_copy(rows[k % 2], out_hbm.at[c, pl.ds(rp, CHUNK)],
                                  gsem[k % 2]).wait()
        pltpu.sync_copy(acc.at[pl.ds(r0, CHUNK)], rows[k % 2])
        pltpu.async_copy(rows[k % 2], out_hbm.at[c, pl.ds(r0, CHUNK)],
                         gsem[k % 2])
    for k in range(nk - 2, nk):
        r0 = r_base + k * CHUNK
        pltpu.make_async_copy(rows[k % 2], out_hbm.at[c, pl.ds(r0, CHUNK)],
                              gsem[k % 2]).wait()
    if do_counts:
        for k in range(nk):
            r0 = r_base + k * CHUNK
            pltpu.sync_copy(cacc.at[pl.ds(r0, CHUNK)], ones_v)
            pltpu.sync_copy(ones_v, cnt_hbm.at[c, pl.ds(r0, CHUNK)])


def _make_segsum(do_counts):
    out_type = [jax.ShapeDtypeStruct((NC, NPAD, CH), jnp.float32)]
    scratch = [pltpu.VMEM_SHARED((NPAD, CH), jnp.float32)]
    if do_counts:
        out_type.append(jax.ShapeDtypeStruct((NC, NPAD, CW), jnp.float32))
        scratch.append(pltpu.VMEM_SHARED((NPAD, CW), jnp.float32))
    scratch += [
        pltpu.VMEM((NCHUNK, CHUNK), jnp.int32),
        pltpu.VMEM((NCHUNK, CHUNK), jnp.int32),
    ]
    if do_counts:
        scratch.append(pltpu.VMEM((CHUNK, CW), jnp.float32))
    scratch += [pltpu.VMEM((CHUNK, CH), jnp.float32) for _ in range(NBUF)]
    scratch += [pltpu.SemaphoreType.DMA for _ in range(2 * NBUF)]
    if do_counts:
        scratch.append(pltpu.SemaphoreType.DMA)
    mesh = plsc.VectorSubcoreMesh(core_axis_name="c", subcore_axis_name="s")
    return pl.kernel(
        functools.partial(_segsum_body, do_counts),
        out_type=tuple(out_type),
        mesh=mesh,
        scratch_types=scratch,
        compiler_params=pltpu.CompilerParams(use_tc_tiling_on_sc=False),
    )


_BM = 1024


def _tc_pre_body(x_ref, w_ref, b_ref, o_ref):
    o_ref[...] = (jnp.dot(x_ref[...], w_ref[...],
                          preferred_element_type=jnp.float32) + b_ref[...])


def _tc_pre(x, w, b):
    grid = (pl.cdiv(N, _BM),)
    return pl.pallas_call(
        _tc_pre_body,
        grid=grid,
        in_specs=[
            pl.BlockSpec((_BM, C), lambda i: (i, 0)),
            pl.BlockSpec((C, C), lambda i: (0, 0)),
            pl.BlockSpec((1, C), lambda i: (0, 0)),
        ],
        out_specs=pl.BlockSpec((_BM, C), lambda i: (i, 0)),
        out_shape=jax.ShapeDtypeStruct((N, C), jnp.float32),
    )(x, w, b)


def _tc1_body(p_ref, cntp_ref, hr_ref, wl_ref, h_ref, hab_ref):
    cnt = cntp_ref[0, :, 0] + cntp_ref[1, :, 0]
    inv = 1.0 / jnp.maximum(cnt, 1.0)
    mean = jnp.concatenate([p_ref[0], p_ref[1]], axis=1) * inv[:, None]
    h = (jnp.dot(mean, wl_ref[...], preferred_element_type=jnp.float32)
         + hr_ref[...])
    h = jnp.maximum(h, 0.0)
    h_ref[...] = h
    hab_ref[0] = h[:, :CH]
    hab_ref[1] = h[:, CH:]


def _tc2_body(p_ref, cntp_ref, hr_ref, wl_ref, o_ref):
    cnt = cntp_ref[0, :, 0] + cntp_ref[1, :, 0]
    inv = 1.0 / jnp.maximum(cnt, 1.0)
    mean = jnp.concatenate([p_ref[0], p_ref[1]], axis=1) * inv[:, None]
    o_ref[...] = (jnp.dot(mean, wl_ref[...], preferred_element_type=jnp.float32)
                  + hr_ref[...])


def _tc_layer1(p, cntp, hr, wl):
    grid = (pl.cdiv(N, _BM),)
    return pl.pallas_call(
        _tc1_body,
        grid=grid,
        in_specs=[
            pl.BlockSpec((NC, _BM, CH), lambda i: (0, i, 0)),
            pl.BlockSpec((NC, _BM, CW), lambda i: (0, i, 0)),
            pl.BlockSpec((_BM, C), lambda i: (i, 0)),
            pl.BlockSpec((C, C), lambda i: (0, 0)),
        ],
        out_specs=[
            pl.BlockSpec((_BM, C), lambda i: (i, 0)),
            pl.BlockSpec((NC, _BM, CH), lambda i: (0, i, 0)),
        ],
        out_shape=[
            jax.ShapeDtypeStruct((N, C), jnp.float32),
            jax.ShapeDtypeStruct((NC, N, CH), jnp.float32),
        ],
    )(p, cntp, hr, wl)


def _tc_layer2(p, cntp, hr, wl):
    grid = (pl.cdiv(N, _BM),)
    return pl.pallas_call(
        _tc2_body,
        grid=grid,
        in_specs=[
            pl.BlockSpec((NC, _BM, CH), lambda i: (0, i, 0)),
            pl.BlockSpec((NC, _BM, CW), lambda i: (0, i, 0)),
            pl.BlockSpec((_BM, C), lambda i: (i, 0)),
            pl.BlockSpec((C, C), lambda i: (0, 0)),
        ],
        out_specs=pl.BlockSpec((_BM, C), lambda i: (i, 0)),
        out_shape=jax.ShapeDtypeStruct((N, C), jnp.float32),
    )(p, cntp, hr, wl)


@jax.jit
def kernel(x, edge_index, Wl1, Wr1, b1, Wl2, Wr2, b2):
    src = edge_index[0].astype(jnp.int32)
    dst = edge_index[1].astype(jnp.int32)
    z2d = jnp.zeros((CHUNK, CH), jnp.float32)
    zc = jnp.zeros((CHUNK, CW), jnp.float32)
    ones = jnp.ones((CHUNK, CW), jnp.float32)

    srcab = jnp.stack([src, src + N]).reshape(NC, NS, NCHUNK, CHUNK)
    dst4 = dst.reshape(NS, NCHUNK, CHUNK)
    xab = jnp.concatenate([x[:, :CH], x[:, CH:]], axis=0)
    hr1 = _tc_pre(x, Wr1, b1.reshape(1, C))
    p1, cntp = _make_segsum(True)(srcab, dst4, xab, z2d, zc, ones)
    h, hab = _tc_layer1(p1, cntp, hr1, Wl1)
    hr2 = _tc_pre(h, Wr2, b2.reshape(1, C))
    (p2,) = _make_segsum(False)(srcab, dst4, hab.reshape(NC * N, CH),
                                z2d, zc, ones)
    out = _tc_layer2(p2, cntp, hr2, Wl2)
    return out


# speedup vs baseline: 11.9891x; 1.1490x over previous
import functools

import jax
import jax.numpy as jnp
from jax import lax
from jax.experimental import pallas as pl
from jax.experimental.pallas import tpu as pltpu
from jax.experimental.pallas import tpu_sc as plsc

N = 10000
C = 128
CH = C // 2
E = 320000
NC, NS = 2, 16
E_PER_TILE = E // NS
CHUNK = 40
NCHUNK = E_PER_TILE // CHUNK
NPAD = 10240
ROWS_PER_TILE = NPAD // NS
CW = 16


NBUF = 10
LAG = NBUF // 2
NSUPER = NCHUNK // NBUF


def _segsum_body(do_counts, ei_hbm, xa_hbm, xb_hbm, z2d_hbm, zc_hbm,
                 ones_hbm, *rest):
    if do_counts:
        out_hbm, cnt_hbm, acc, cacc, src_a, dst_a, ones_v, *rows = rest
    else:
        out_hbm, acc, src_a, dst_a, *rows = rest
        cnt_hbm = cacc = ones_v = None
    rows, sems = rows[:NBUF], rows[NBUF:]
    gsem, ssem = sems[:NBUF], sems[NBUF:2 * NBUF]
    csem = sems[2 * NBUF] if do_counts else None

    c = lax.axis_index("c")
    s = lax.axis_index("s")
    r_base = pl.multiple_of(s * ROWS_PER_TILE, ROWS_PER_TILE)

    pltpu.sync_copy(ei_hbm.at[0, s], src_a)
    pltpu.sync_copy(ei_hbm.at[1, s], dst_a)

    def start_gather(idx, buf, sem):
        @pl.when(c == 0)
        def _():
            pltpu.async_copy(xa_hbm.at[idx], buf, sem)

        @pl.when(c == 1)
        def _():
            pltpu.async_copy(xb_hbm.at[idx], buf, sem)

    pltpu.sync_copy(z2d_hbm, rows[0])
    if do_counts:
        pltpu.sync_copy(zc_hbm, ones_v)
    for k in range(ROWS_PER_TILE // CHUNK):
        pltpu.async_copy(rows[0], acc.at[pl.ds(r_base + k * CHUNK, CHUNK)],
                         ssem[0])
        if do_counts:
            pltpu.async_copy(ones_v,
                             cacc.at[pl.ds(r_base + k * CHUNK, CHUNK)],
                             ssem[1])
    for k in range(ROWS_PER_TILE // CHUNK):
        pltpu.make_async_copy(rows[0], acc.at[pl.ds(r_base, CHUNK)],
                              ssem[0]).wait()
        if do_counts:
            pltpu.make_async_copy(ones_v, cacc.at[pl.ds(r_base, CHUNK)],
                                  ssem[1]).wait()
    if do_counts:
        pltpu.sync_copy(ones_hbm, ones_v)

    for b in range(LAG):
        start_gather(src_a.at[b], rows[b], gsem[b])
    plsc.subcore_barrier()

    def super_body(J, carry):
        for b in range(NBUF):
            j = J * NBUF + b
            pltpu.make_async_copy(xa_hbm.at[src_a.at[j]], rows[b],
                                  gsem[b]).wait()
            pltpu.async_copy(rows[b], acc.at[dst_a.at[j]], ssem[b], add=True)
            if do_counts:
                w0 = c * (NSUPER // 2)

                @pl.when((J >= w0) & (J < w0 + NSUPER // 2))
                def _():
                    if b == 0:
                        @pl.when(J > w0)
                        def _():
                            pltpu.make_async_copy(
                                ones_v, cacc.at[dst_a.at[j]], csem).wait()
                    else:
                        pltpu.make_async_copy(
                            ones_v, cacc.at[dst_a.at[j]], csem).wait()
                    pltpu.async_copy(ones_v, cacc.at[dst_a.at[j]], csem,
                                     add=True)
            bp = (b + LAG) % NBUF
            jp = j - LAG
            jn = j + LAG

            def drain_and_refill():
                pltpu.make_async_copy(rows[bp], acc.at[dst_a.at[jp]],
                                      ssem[bp]).wait()
                start_gather(src_a.at[jn], rows[bp], gsem[bp])

            if b < LAG:
                @pl.when(J > 0)
                def _():
                    drain_and_refill()

                @pl.when(J == 0)
                def _():
                    start_gather(src_a.at[jn], rows[bp], gsem[bp])
            else:
                @pl.when(J < NSUPER - 1)
                def _():
                    drain_and_refill()

                @pl.when(J == NSUPER - 1)
                def _():
                    pltpu.make_async_copy(rows[bp], acc.at[dst_a.at[jp]],
                                          ssem[bp]).wait()
        return carry

    lax.fori_loop(0, NSUPER, super_body, 0)

    for b in range(LAG, NBUF):
        pltpu.make_async_copy(rows[b], acc.at[dst_a.at[NCHUNK - LAG + b - LAG]],
                              ssem[b]).wait()
    if do_counts:
        pltpu.make_async_copy(ones_v, cacc.at[dst_a.at[0]], csem).wait()
    plsc.subcore_barrier()

    nk = ROWS_PER_TILE // CHUNK
    for k in range(nk):
        r0 = r_base + k * CHUNK
        if k >= 2:
            rp = r_base + (k - 2) * CHUNK
            pltpu.make_async_copy(rows[k % 2], out_hbm.at[c, pl.ds(rp, CHUNK)],
                                  gsem[k % 2]).wait()
        pltpu.sync_copy(acc.at[pl.ds(r0, CHUNK)], rows[k % 2])
        pltpu.async_copy(rows[k % 2], out_hbm.at[c, pl.ds(r0, CHUNK)],
                         gsem[k % 2])
    for k in range(nk - 2, nk):
        r0 = r_base + k * CHUNK
        pltpu.make_async_copy(rows[k % 2], out_hbm.at[c, pl.ds(r0, CHUNK)],
                              gsem[k % 2]).wait()
    if do_counts:
        for k in range(nk):
            r0 = r_base + k * CHUNK
            pltpu.sync_copy(cacc.at[pl.ds(r0, CHUNK)], ones_v)
            pltpu.sync_copy(ones_v, cnt_hbm.at[c, pl.ds(r0, CHUNK)])


def _make_segsum(do_counts):
    out_type = [jax.ShapeDtypeStruct((NC, NPAD, CH), jnp.float32)]
    scratch = [pltpu.VMEM_SHARED((NPAD, CH), jnp.float32)]
    if do_counts:
        out_type.append(jax.ShapeDtypeStruct((NC, NPAD, CW), jnp.float32))
        scratch.append(pltpu.VMEM_SHARED((NPAD, CW), jnp.float32))
    scratch += [
        pltpu.VMEM((NCHUNK, CHUNK), jnp.int32),
        pltpu.VMEM((NCHUNK, CHUNK), jnp.int32),
    ]
    if do_counts:
        scratch.append(pltpu.VMEM((CHUNK, CW), jnp.float32))
    scratch += [pltpu.VMEM((CHUNK, CH), jnp.float32) for _ in range(NBUF)]
    scratch += [pltpu.SemaphoreType.DMA for _ in range(2 * NBUF)]
    if do_counts:
        scratch.append(pltpu.SemaphoreType.DMA)
    mesh = plsc.VectorSubcoreMesh(core_axis_name="c", subcore_axis_name="s")
    return pl.kernel(
        functools.partial(_segsum_body, do_counts),
        out_type=tuple(out_type),
        mesh=mesh,
        scratch_types=scratch,
        compiler_params=pltpu.CompilerParams(use_tc_tiling_on_sc=False),
    )


_BM = 1024


def _tc_pre_body(x_ref, w_ref, b_ref, o_ref):
    o_ref[...] = (jnp.dot(x_ref[...], w_ref[...],
                          preferred_element_type=jnp.float32) + b_ref[...])


def _tc_pre(x, w, b):
    grid = (pl.cdiv(N, _BM),)
    return pl.pallas_call(
        _tc_pre_body,
        grid=grid,
        in_specs=[
            pl.BlockSpec((_BM, C), lambda i: (i, 0)),
            pl.BlockSpec((C, C), lambda i: (0, 0)),
            pl.BlockSpec((1, C), lambda i: (0, 0)),
        ],
        out_specs=pl.BlockSpec((_BM, C), lambda i: (i, 0)),
        out_shape=jax.ShapeDtypeStruct((N, C), jnp.float32),
    )(x, w, b)


def _tc1_body(p_ref, cntp_ref, hr_ref, wl_ref, ha_ref, hb_ref):
    cnt = cntp_ref[0, :, 0] + cntp_ref[1, :, 0]
    inv = 1.0 / jnp.maximum(cnt, 1.0)
    mean = jnp.concatenate([p_ref[0], p_ref[1]], axis=1) * inv[:, None]
    h = (jnp.dot(mean, wl_ref[...], preferred_element_type=jnp.float32)
         + hr_ref[...])
    h = jnp.maximum(h, 0.0)
    ha_ref[...] = h[:, :CH]
    hb_ref[...] = h[:, CH:]


def _tc_pre2_body(ha_ref, hb_ref, w_ref, b_ref, o_ref):
    o_ref[...] = (jnp.dot(ha_ref[...], w_ref[:CH],
                          preferred_element_type=jnp.float32)
                  + jnp.dot(hb_ref[...], w_ref[CH:],
                            preferred_element_type=jnp.float32)
                  + b_ref[...])


def _tc_pre2(ha, hb, w, b):
    grid = (pl.cdiv(N, _BM),)
    return pl.pallas_call(
        _tc_pre2_body,
        grid=grid,
        in_specs=[
            pl.BlockSpec((_BM, CH), lambda i: (i, 0)),
            pl.BlockSpec((_BM, CH), lambda i: (i, 0)),
            pl.BlockSpec((C, C), lambda i: (0, 0)),
            pl.BlockSpec((1, C), lambda i: (0, 0)),
        ],
        out_specs=pl.BlockSpec((_BM, C), lambda i: (i, 0)),
        out_shape=jax.ShapeDtypeStruct((N, C), jnp.float32),
    )(ha, hb, w, b)


def _tc2_body(p_ref, cntp_ref, hr_ref, wl_ref, o_ref):
    cnt = cntp_ref[0, :, 0] + cntp_ref[1, :, 0]
    inv = 1.0 / jnp.maximum(cnt, 1.0)
    mean = jnp.concatenate([p_ref[0], p_ref[1]], axis=1) * inv[:, None]
    o_ref[...] = (jnp.dot(mean, wl_ref[...], preferred_element_type=jnp.float32)
                  + hr_ref[...])


def _tc_layer1(p, cntp, hr, wl):
    grid = (pl.cdiv(N, _BM),)
    return pl.pallas_call(
        _tc1_body,
        grid=grid,
        in_specs=[
            pl.BlockSpec((NC, _BM, CH), lambda i: (0, i, 0)),
            pl.BlockSpec((NC, _BM, CW), lambda i: (0, i, 0)),
            pl.BlockSpec((_BM, C), lambda i: (i, 0)),
            pl.BlockSpec((C, C), lambda i: (0, 0)),
        ],
        out_specs=[
            pl.BlockSpec((_BM, CH), lambda i: (i, 0)),
            pl.BlockSpec((_BM, CH), lambda i: (i, 0)),
        ],
        out_shape=[
            jax.ShapeDtypeStruct((N, CH), jnp.float32),
            jax.ShapeDtypeStruct((N, CH), jnp.float32),
        ],
    )(p, cntp, hr, wl)


def _tc_layer2(p, cntp, hr, wl):
    grid = (pl.cdiv(N, _BM),)
    return pl.pallas_call(
        _tc2_body,
        grid=grid,
        in_specs=[
            pl.BlockSpec((NC, _BM, CH), lambda i: (0, i, 0)),
            pl.BlockSpec((NC, _BM, CW), lambda i: (0, i, 0)),
            pl.BlockSpec((_BM, C), lambda i: (i, 0)),
            pl.BlockSpec((C, C), lambda i: (0, 0)),
        ],
        out_specs=pl.BlockSpec((_BM, C), lambda i: (i, 0)),
        out_shape=jax.ShapeDtypeStruct((N, C), jnp.float32),
    )(p, cntp, hr, wl)


@jax.jit
def kernel(x, edge_index, Wl1, Wr1, b1, Wl2, Wr2, b2):
    src = edge_index[0].astype(jnp.int32)
    dst = edge_index[1].astype(jnp.int32)
    z2d = jnp.zeros((CHUNK, CH), jnp.float32)
    zc = jnp.zeros((CHUNK, CW), jnp.float32)
    ones = jnp.ones((CHUNK, CW), jnp.float32)

    ei4 = jnp.stack([src, dst]).reshape(2, NS, NCHUNK, CHUNK)
    xa = x[:, :CH]
    xb = x[:, CH:]
    hr1 = _tc_pre(x, Wr1, b1.reshape(1, C))
    p1, cntp = _make_segsum(True)(ei4, xa, xb, z2d, zc, ones)
    ha, hb = _tc_layer1(p1, cntp, hr1, Wl1)
    hr2 = _tc_pre2(ha, hb, Wr2, b2.reshape(1, C))
    (p2,) = _make_segsum(False)(ei4, ha, hb, z2d, zc, ones)
    out = _tc_layer2(p2, cntp, hr2, Wl2)
    return out


# speedup vs baseline: 12.7622x; 1.0645x over previous
import functools

import jax
import jax.numpy as jnp
from jax import lax
from jax.experimental import pallas as pl
from jax.experimental.pallas import tpu as pltpu
from jax.experimental.pallas import tpu_sc as plsc

N = 10000
C = 128
CH = C // 2
E = 320000
NC, NS = 2, 16
E_PER_TILE = E // NS
CHUNK = 80
NCHUNK = E_PER_TILE // CHUNK
NPAD = 10240
ROWS_PER_TILE = NPAD // NS
CW = 16


NBUF = 10
LAG = NBUF // 2
NSTAGE = 5
CPS = NCHUNK // NSTAGE
SPS = CPS // NBUF
NSUPER = NCHUNK // NBUF


def _segsum_body(do_counts, ei_hbm, xa_hbm, xb_hbm, z2d_hbm, zc_hbm,
                 ones_hbm, *rest):
    if do_counts:
        out_hbm, cnt_hbm, acc, cacc, src_a, dst_a, ones_v, *rows = rest
    else:
        out_hbm, acc, src_a, dst_a, *rows = rest
        cnt_hbm = cacc = ones_v = None
    rows, sems = rows[:NBUF], rows[NBUF:]
    gsem, ssem = sems[:NBUF], sems[NBUF:2 * NBUF]
    isem_s, isem_d = sems[2 * NBUF], sems[2 * NBUF + 1]
    csem = sems[2 * NBUF + 2] if do_counts else None

    c = lax.axis_index("c")
    s = lax.axis_index("s")
    r_base = pl.multiple_of(s * ROWS_PER_TILE, ROWS_PER_TILE)

    pltpu.sync_copy(ei_hbm.at[0, s, 0], src_a.at[0])
    pltpu.sync_copy(ei_hbm.at[1, s, 0], dst_a.at[0])

    def start_gather(idx, buf, sem):
        @pl.when(c == 0)
        def _():
            pltpu.async_copy(xa_hbm.at[idx], buf, sem)

        @pl.when(c == 1)
        def _():
            pltpu.async_copy(xb_hbm.at[idx], buf, sem)

    pltpu.sync_copy(z2d_hbm, rows[0])
    if do_counts:
        pltpu.sync_copy(zc_hbm, ones_v)
    for k in range(ROWS_PER_TILE // CHUNK):
        pltpu.async_copy(rows[0], acc.at[pl.ds(r_base + k * CHUNK, CHUNK)],
                         ssem[0])
        if do_counts:
            pltpu.async_copy(ones_v,
                             cacc.at[pl.ds(r_base + k * CHUNK, CHUNK)],
                             ssem[1])
    for k in range(ROWS_PER_TILE // CHUNK):
        pltpu.make_async_copy(rows[0], acc.at[pl.ds(r_base, CHUNK)],
                              ssem[0]).wait()
        if do_counts:
            pltpu.make_async_copy(ones_v, cacc.at[pl.ds(r_base, CHUNK)],
                                  ssem[1]).wait()
    if do_counts:
        pltpu.sync_copy(ones_hbm, ones_v)

    for b in range(LAG):
        start_gather(src_a.at[0, b], rows[b], gsem[b])
    plsc.subcore_barrier()

    def super_body(S, carry):
        T = S // SPS
        JJ = S % SPS
        half = T % 2
        nhalf = (T + 1) % 2
        for b in range(NBUF):
            j = S * NBUF + b
            row = JJ * NBUF + b
            if b == 0:
                @pl.when((JJ == 0) & (T < NSTAGE - 1))
                def _():
                    pltpu.async_copy(ei_hbm.at[0, s, T + 1], src_a.at[nhalf],
                                     isem_s)
                    pltpu.async_copy(ei_hbm.at[1, s, T + 1], dst_a.at[nhalf],
                                     isem_d)
            pltpu.make_async_copy(xa_hbm.at[src_a.at[0, 0]], rows[b],
                                  gsem[b]).wait()
            pltpu.async_copy(rows[b], acc.at[dst_a.at[half, row]], ssem[b],
                             add=True)
            if do_counts:
                j0 = c * (NCHUNK // 2)
                active = (j >= j0) & (j < j0 + NCHUNK // 2)

                @pl.when(active & (j > j0))
                def _():
                    pltpu.make_async_copy(
                        ones_v, cacc.at[dst_a.at[0, 0]], csem).wait()

                @pl.when(active)
                def _():
                    pltpu.async_copy(ones_v, cacc.at[dst_a.at[half, row]],
                                     csem, add=True)
            bp = (b + LAG) % NBUF

            def drain():
                pltpu.make_async_copy(rows[bp], acc.at[dst_a.at[0, 0]],
                                      ssem[bp]).wait()

            if b < LAG:
                @pl.when(S > 0)
                def _():
                    drain()

                start_gather(src_a.at[half, row + LAG], rows[bp], gsem[bp])
            else:
                @pl.when(S < NSUPER - 1)
                def _():
                    drain()

                    @pl.when(JJ < SPS - 1)
                    def _():
                        start_gather(src_a.at[half, row + LAG], rows[bp],
                                     gsem[bp])

                @pl.when(S == NSUPER - 1)
                def _():
                    drain()
                if b == LAG:
                    @pl.when((JJ == SPS - 1) & (T < NSTAGE - 1))
                    def _():
                        pltpu.make_async_copy(ei_hbm.at[0, s, 0],
                                              src_a.at[0], isem_s).wait()
                        pltpu.make_async_copy(ei_hbm.at[1, s, 0],
                                              dst_a.at[0], isem_d).wait()

                @pl.when((JJ == SPS - 1) & (T < NSTAGE - 1))
                def _():
                    start_gather(src_a.at[nhalf, b - LAG], rows[bp], gsem[bp])
        return carry

    lax.fori_loop(0, NSUPER, super_body, 0)

    for b in range(LAG, NBUF):
        pltpu.make_async_copy(rows[b], acc.at[dst_a.at[0, 0]], ssem[b]).wait()
    if do_counts:
        pltpu.make_async_copy(ones_v, cacc.at[dst_a.at[0, 0]], csem).wait()
    plsc.subcore_barrier()

    nk = ROWS_PER_TILE // CHUNK
    for k in range(nk):
        r0 = r_base + k * CHUNK
        if k >= 2:
            rp = r_base + (k - 2) * CHUNK
            pltpu.make_async_copy(rows[k % 2], out_hbm.at[c, pl.ds(rp, CHUNK)],
                                  gsem[k % 2]).wait()
        pltpu.sync_copy(acc.at[pl.ds(r0, CHUNK)], rows[k % 2])
        pltpu.async_copy(rows[k % 2], out_hbm.at[c, pl.ds(r0, CHUNK)],
                         gsem[k % 2])
    for k in range(nk - 2, nk):
        r0 = r_base + k * CHUNK
        pltpu.make_async_copy(rows[k % 2], out_hbm.at[c, pl.ds(r0, CHUNK)],
                              gsem[k % 2]).wait()
    if do_counts:
        for k in range(nk):
            r0 = r_base + k * CHUNK
            pltpu.sync_copy(cacc.at[pl.ds(r0, CHUNK)], ones_v)
            pltpu.sync_copy(ones_v, cnt_hbm.at[c, pl.ds(r0, CHUNK)])


def _make_segsum(do_counts):
    out_type = [jax.ShapeDtypeStruct((NC, NPAD, CH), jnp.float32)]
    scratch = [pltpu.VMEM_SHARED((NPAD, CH), jnp.float32)]
    if do_counts:
        out_type.append(jax.ShapeDtypeStruct((NC, NPAD, CW), jnp.float32))
        scratch.append(pltpu.VMEM_SHARED((NPAD, CW), jnp.float32))
    scratch += [
        pltpu.VMEM((2, CPS, CHUNK), jnp.int32),
        pltpu.VMEM((2, CPS, CHUNK), jnp.int32),
    ]
    if do_counts:
        scratch.append(pltpu.VMEM((CHUNK, CW), jnp.float32))
    scratch += [pltpu.VMEM((CHUNK, CH), jnp.float32) for _ in range(NBUF)]
    scratch += [pltpu.SemaphoreType.DMA for _ in range(2 * NBUF + 2)]
    if do_counts:
        scratch.append(pltpu.SemaphoreType.DMA)
    mesh = plsc.VectorSubcoreMesh(core_axis_name="c", subcore_axis_name="s")
    return pl.kernel(
        functools.partial(_segsum_body, do_counts),
        out_type=tuple(out_type),
        mesh=mesh,
        scratch_types=scratch,
        compiler_params=pltpu.CompilerParams(use_tc_tiling_on_sc=False),
    )


_BM = 1024


def _tc_pre_body(x_ref, w_ref, b_ref, o_ref):
    o_ref[...] = (jnp.dot(x_ref[...], w_ref[...],
                          preferred_element_type=jnp.float32) + b_ref[...])


def _tc_pre(x, w, b):
    grid = (pl.cdiv(N, _BM),)
    return pl.pallas_call(
        _tc_pre_body,
        grid=grid,
        in_specs=[
            pl.BlockSpec((_BM, C), lambda i: (i, 0)),
            pl.BlockSpec((C, C), lambda i: (0, 0)),
            pl.BlockSpec((1, C), lambda i: (0, 0)),
        ],
        out_specs=pl.BlockSpec((_BM, C), lambda i: (i, 0)),
        out_shape=jax.ShapeDtypeStruct((N, C), jnp.float32),
    )(x, w, b)


def _tc1_body(p_ref, cntp_ref, hr_ref, wl_ref, ha_ref, hb_ref):
    cnt = cntp_ref[0, :, 0] + cntp_ref[1, :, 0]
    inv = 1.0 / jnp.maximum(cnt, 1.0)
    mean = jnp.concatenate([p_ref[0], p_ref[1]], axis=1) * inv[:, None]
    h = (jnp.dot(mean, wl_ref[...], preferred_element_type=jnp.float32)
         + hr_ref[...])
    h = jnp.maximum(h, 0.0)
    ha_ref[...] = h[:, :CH]
    hb_ref[...] = h[:, CH:]


def _tc_pre2_body(ha_ref, hb_ref, w_ref, b_ref, o_ref):
    o_ref[...] = (jnp.dot(ha_ref[...], w_ref[:CH],
                          preferred_element_type=jnp.float32)
                  + jnp.dot(hb_ref[...], w_ref[CH:],
                            preferred_element_type=jnp.float32)
                  + b_ref[...])


def _tc_pre2(ha, hb, w, b):
    grid = (pl.cdiv(N, _BM),)
    return pl.pallas_call(
        _tc_pre2_body,
        grid=grid,
        in_specs=[
            pl.BlockSpec((_BM, CH), lambda i: (i, 0)),
            pl.BlockSpec((_BM, CH), lambda i: (i, 0)),
            pl.BlockSpec((C, C), lambda i: (0, 0)),
            pl.BlockSpec((1, C), lambda i: (0, 0)),
        ],
        out_specs=pl.BlockSpec((_BM, C), lambda i: (i, 0)),
        out_shape=jax.ShapeDtypeStruct((N, C), jnp.float32),
    )(ha, hb, w, b)


def _tc2_body(p_ref, cntp_ref, hr_ref, wl_ref, o_ref):
    cnt = cntp_ref[0, :, 0] + cntp_ref[1, :, 0]
    inv = 1.0 / jnp.maximum(cnt, 1.0)
    mean = jnp.concatenate([p_ref[0], p_ref[1]], axis=1) * inv[:, None]
    o_ref[...] = (jnp.dot(mean, wl_ref[...], preferred_element_type=jnp.float32)
                  + hr_ref[...])


def _tc_layer1(p, cntp, hr, wl):
    grid = (pl.cdiv(N, _BM),)
    return pl.pallas_call(
        _tc1_body,
        grid=grid,
        in_specs=[
            pl.BlockSpec((NC, _BM, CH), lambda i: (0, i, 0)),
            pl.BlockSpec((NC, _BM, CW), lambda i: (0, i, 0)),
            pl.BlockSpec((_BM, C), lambda i: (i, 0)),
            pl.BlockSpec((C, C), lambda i: (0, 0)),
        ],
        out_specs=[
            pl.BlockSpec((_BM, CH), lambda i: (i, 0)),
            pl.BlockSpec((_BM, CH), lambda i: (i, 0)),
        ],
        out_shape=[
            jax.ShapeDtypeStruct((N, CH), jnp.float32),
            jax.ShapeDtypeStruct((N, CH), jnp.float32),
        ],
    )(p, cntp, hr, wl)


def _tc_layer2(p, cntp, hr, wl):
    grid = (pl.cdiv(N, _BM),)
    return pl.pallas_call(
        _tc2_body,
        grid=grid,
        in_specs=[
            pl.BlockSpec((NC, _BM, CH), lambda i: (0, i, 0)),
            pl.BlockSpec((NC, _BM, CW), lambda i: (0, i, 0)),
            pl.BlockSpec((_BM, C), lambda i: (i, 0)),
            pl.BlockSpec((C, C), lambda i: (0, 0)),
        ],
        out_specs=pl.BlockSpec((_BM, C), lambda i: (i, 0)),
        out_shape=jax.ShapeDtypeStruct((N, C), jnp.float32),
    )(p, cntp, hr, wl)


@jax.jit
def kernel(x, edge_index, Wl1, Wr1, b1, Wl2, Wr2, b2):
    src = edge_index[0].astype(jnp.int32)
    dst = edge_index[1].astype(jnp.int32)
    z2d = jnp.zeros((CHUNK, CH), jnp.float32)
    zc = jnp.zeros((CHUNK, CW), jnp.float32)
    ones = jnp.ones((CHUNK, CW), jnp.float32)

    ei4 = jnp.stack([src, dst]).reshape(2, NS, NSTAGE, CPS, CHUNK)
    xa = x[:, :CH]
    xb = x[:, CH:]
    hr1 = _tc_pre(x, Wr1, b1.reshape(1, C))
    p1, cntp = _make_segsum(True)(ei4, xa, xb, z2d, zc, ones)
    ha, hb = _tc_layer1(p1, cntp, hr1, Wl1)
    hr2 = _tc_pre2(ha, hb, Wr2, b2.reshape(1, C))
    (p2,) = _make_segsum(False)(ei4, ha, hb, z2d, zc, ones)
    out = _tc_layer2(p2, cntp, hr2, Wl2)
    return out
